# baseline (speedup 1.0000x reference)
import functools

import jax
import jax.numpy as jnp
from jax import lax
from jax.experimental import pallas as pl
from jax.experimental.pallas import tpu as pltpu
from jax.experimental.pallas import tpu_sc as plsc

_NC = 2
_NS = 16
_NW = _NC * _NS

_CHUNK = 640
_KV = 32768


def _tpack_block(t_ref, o_ref):
    embed = t_ref.shape[0]
    o_ref[:, :embed] = t_ref[...].T


def _tc_transpose_pack(table_t):
    embed, vocab = table_t.shape
    return pl.pallas_call(
        _tpack_block,
        grid=(pl.cdiv(vocab, _KV),),
        in_specs=[pl.BlockSpec((embed, _KV), lambda i: (0, i))],
        out_specs=pl.BlockSpec((_KV, 2 * embed), lambda i: (i, 0)),
        out_shape=jax.ShapeDtypeStruct((vocab, 2 * embed), jnp.float32),
    )(table_t)


def _sc_gather(table_wide, idx):
    n, = idx.shape
    d = table_wide.shape[1]
    b_per_w = n // _NW
    n_chunks = b_per_w // _CHUNK
    mesh = plsc.VectorSubcoreMesh(core_axis_name="c", subcore_axis_name="s")

    @functools.partial(
        pl.kernel,
        mesh=mesh,
        out_type=jax.ShapeDtypeStruct((n, d), jnp.float32),
        scratch_types=[
            pltpu.VMEM((b_per_w,), jnp.int32),
            pltpu.VMEM((_CHUNK, d), jnp.float32),
            pltpu.SemaphoreType.DMA,
        ],
    )
    def gather_kernel(table_hbm, idx_hbm, out_hbm, idx_v, rows_v, sem):
        wid = lax.axis_index("s") * _NC + lax.axis_index("c")
        base = wid * b_per_w
        pltpu.sync_copy(idx_hbm.at[pl.ds(base, b_per_w)], idx_v)

        @pl.loop(0, n_chunks)
        def _(c):
            off = c * _CHUNK
            pltpu.async_copy(
                table_hbm.at[idx_v.at[pl.ds(off, _CHUNK)]], rows_v, sem
            ).wait()
            pltpu.sync_copy(rows_v, out_hbm.at[pl.ds(base + off, _CHUNK)])

    return gather_kernel(table_wide, idx)


def _mlp_block(r0, r1, r2, r3, r4, w1_ref, b1_ref, w2_ref, b2_ref, o_ref):
    h = b1_ref[...]
    embed = w1_ref.shape[1]
    for w, rw in enumerate((r0, r1, r2, r3, r4)):
        h = h + jnp.dot(rw[:, :embed], w1_ref[w],
                        preferred_element_type=jnp.float32)
    h = jnp.maximum(h, 0.0)
    o_ref[...] = (
        jnp.dot(h, w2_ref[...], preferred_element_type=jnp.float32) + b2_ref[...]
    )


def _tc_mlp(rows, w1s, b1, w2, b2, batch, block_b=2048):
    window, embed, hidden = w1s.shape
    out_dim = w2.shape[1]
    nb = batch // block_b
    row_specs = [
        pl.BlockSpec((block_b, 2 * embed), functools.partial(
            lambda w, i: (w * nb + i, 0), w))
        for w in range(window)
    ]
    return pl.pallas_call(
        _mlp_block,
        grid=(nb,),
        in_specs=row_specs + [
            pl.BlockSpec((window, embed, hidden), lambda i: (0, 0, 0)),
            pl.BlockSpec((1, hidden), lambda i: (0, 0)),
            pl.BlockSpec((hidden, out_dim), lambda i: (0, 0)),
            pl.BlockSpec((1, out_dim), lambda i: (0, 0)),
        ],
        out_specs=pl.BlockSpec((block_b, out_dim), lambda i: (i, 0)),
        out_shape=jax.ShapeDtypeStruct((batch, out_dim), jnp.float32),
    )(*([rows] * window), w1s, b1, w2, b2)


def kernel(word_indices, table, W1, b1, W2, b2):
    batch, window = word_indices.shape
    vocab, embed = table.shape

    table_wide = _tc_transpose_pack(table.T)

    idx2d = word_indices.T.astype(jnp.int32)
    half = batch // 2
    w1s = W1.reshape(window, embed, -1)
    b1r, b2r = b1.reshape(1, -1), b2.reshape(1, -1)

    outs = []
    rows_h = [
        _sc_gather(table_wide, idx2d[:, h * half:(h + 1) * half].reshape(-1))
        for h in range(2)
    ]
    for h in range(2):
        outs.append(_tc_mlp(rows_h[h], w1s, b1r, W2, b2r, half))
    return jnp.concatenate(outs, axis=0)


# speedup vs baseline: 1.0154x; 1.0154x over previous
import functools

import jax
import jax.numpy as jnp
from jax import lax
from jax.experimental import pallas as pl
from jax.experimental.pallas import tpu as pltpu
from jax.experimental.pallas import tpu_sc as plsc

_NC = 2
_NS = 16
_NW = _NC * _NS

_CHUNK = 640
_KV = 32768


def _tpack_block(t_ref, o_ref):
    embed = t_ref.shape[0]
    o_ref[:, :embed] = t_ref[...].T


def _tc_transpose_pack(table_t):
    embed, vocab = table_t.shape
    return pl.pallas_call(
        _tpack_block,
        grid=(pl.cdiv(vocab, _KV),),
        in_specs=[pl.BlockSpec((embed, _KV), lambda i: (0, i))],
        out_specs=pl.BlockSpec((_KV, 2 * embed), lambda i: (i, 0)),
        out_shape=jax.ShapeDtypeStruct((vocab, 2 * embed), jnp.float32),
    )(table_t)


def _sc_gather(table_wide, idx):
    n, = idx.shape
    d = table_wide.shape[1]
    b_per_w = n // _NW
    n_chunks = b_per_w // _CHUNK
    mesh = plsc.VectorSubcoreMesh(core_axis_name="c", subcore_axis_name="s")

    @functools.partial(
        pl.kernel,
        mesh=mesh,
        out_type=jax.ShapeDtypeStruct((n, d), jnp.float32),
        scratch_types=[
            pltpu.VMEM((b_per_w,), jnp.int32),
            pltpu.VMEM((_CHUNK, d), jnp.float32),
            pltpu.SemaphoreType.DMA,
        ],
    )
    def gather_kernel(table_hbm, idx_hbm, out_hbm, idx_v, rows_v, sem):
        wid = lax.axis_index("s") * _NC + lax.axis_index("c")
        base = wid * b_per_w
        pltpu.sync_copy(idx_hbm.at[pl.ds(base, b_per_w)], idx_v)

        @pl.loop(0, n_chunks)
        def _(c):
            off = c * _CHUNK
            pltpu.async_copy(
                table_hbm.at[idx_v.at[pl.ds(off, _CHUNK)]], rows_v, sem
            ).wait()
            pltpu.sync_copy(rows_v, out_hbm.at[pl.ds(base + off, _CHUNK)])

    return gather_kernel(table_wide, idx)


def _mlp_block(r0, r1, r2, r3, r4, w1_ref, b1_ref, w2_ref, b2_ref, o_ref):
    h = b1_ref[...]
    embed = w1_ref.shape[1]
    for w, rw in enumerate((r0, r1, r2, r3, r4)):
        h = h + jnp.dot(rw[:, :embed], w1_ref[w],
                        preferred_element_type=jnp.float32)
    h = jnp.maximum(h, 0.0)
    o_ref[...] = (
        jnp.dot(h, w2_ref[...], preferred_element_type=jnp.float32) + b2_ref[...]
    )


def _tc_mlp(rows, w1s, b1, w2, b2, batch, block_b=2048):
    window, embed, hidden = w1s.shape
    out_dim = w2.shape[1]
    nb = batch // block_b
    row_specs = [
        pl.BlockSpec((block_b, 2 * embed), functools.partial(
            lambda w, i: (w * nb + i, 0), w))
        for w in range(window)
    ]
    return pl.pallas_call(
        _mlp_block,
        grid=(nb,),
        in_specs=row_specs + [
            pl.BlockSpec((window, embed, hidden), lambda i: (0, 0, 0)),
            pl.BlockSpec((1, hidden), lambda i: (0, 0)),
            pl.BlockSpec((hidden, out_dim), lambda i: (0, 0)),
            pl.BlockSpec((1, out_dim), lambda i: (0, 0)),
        ],
        out_specs=pl.BlockSpec((block_b, out_dim), lambda i: (i, 0)),
        out_shape=jax.ShapeDtypeStruct((batch, out_dim), jnp.float32),
    )(*([rows] * window), w1s, b1, w2, b2)


def kernel(word_indices, table, W1, b1, W2, b2):
    batch, window = word_indices.shape
    vocab, embed = table.shape

    table_wide = _tc_transpose_pack(table.T)

    idx_wmajor = word_indices.T.reshape(-1).astype(jnp.int32)

    rows = _sc_gather(table_wide, idx_wmajor)

    w1s = W1.reshape(window, embed, -1)
    return _tc_mlp(rows, w1s, b1.reshape(1, -1), W2, b2.reshape(1, -1), batch)
